# baseline (speedup 1.0000x reference)
import jax
import jax.numpy as jnp
from jax.experimental import pallas as pl
from jax.experimental.pallas import tpu as pltpu

_N = 10000
_BLK = 1000


def _mlp_block(x_ref, w1_ref, b1_ref, w2_ref, b2_ref, w3_ref, b3_ref, out_ref):
    h = jnp.dot(x_ref[...], w1_ref[...], preferred_element_type=jnp.float32)
    h = jnp.maximum(h + b1_ref[...], 0.0)
    h = jnp.dot(h, w2_ref[...], preferred_element_type=jnp.float32)
    h = jnp.maximum(h + b2_ref[...], 0.0)
    logits = jnp.dot(h, w3_ref[...], preferred_element_type=jnp.float32)
    logits = logits + b3_ref[...]
    m = jnp.max(logits, axis=1, keepdims=True)
    e = jnp.exp(logits - m)
    out_ref[...] = e / jnp.sum(e, axis=1, keepdims=True)


def kernel(x, edge_index, edge_weight, W1, b1, W2, b2, W3, b3):
    del edge_index, edge_weight
    f_in = x.shape[1]
    c = W2.shape[0]
    n_cls = W3.shape[1]

    grid = (_N // _BLK,)
    fixed = lambda i: (0, 0)
    fixed1 = lambda i: (0,)
    out = pl.pallas_call(
        _mlp_block,
        grid=grid,
        in_specs=[
            pl.BlockSpec((_BLK, f_in), lambda i: (i, 0)),
            pl.BlockSpec((f_in, c), fixed),
            pl.BlockSpec((c,), fixed1),
            pl.BlockSpec((c, c), fixed),
            pl.BlockSpec((c,), fixed1),
            pl.BlockSpec((c, n_cls), fixed),
            pl.BlockSpec((n_cls,), fixed1),
        ],
        out_specs=pl.BlockSpec((_BLK, n_cls), lambda i: (i, 0)),
        out_shape=jax.ShapeDtypeStruct((_N, n_cls), jnp.float32),
        compiler_params=pltpu.CompilerParams(
            dimension_semantics=("parallel",),
        ),
    )(x, W1, b1, W2, b2, W3, b3)
    return out


# speedup vs baseline: 1.9219x; 1.9219x over previous
import jax
import jax.numpy as jnp
from jax.experimental import pallas as pl
from jax.experimental.pallas import tpu as pltpu

_N = 10000
_BLK = 2560


def _mlp_block(x_ref, w1_ref, b1_ref, w2_ref, b2_ref, w3t_ref, b3_ref, out_ref):
    h = jnp.dot(x_ref[...], w1_ref[...], preferred_element_type=jnp.float32)
    h = jnp.maximum(h + b1_ref[...], 0.0)
    h = jnp.dot(h, w2_ref[...], preferred_element_type=jnp.float32)
    h = jnp.maximum(h + b2_ref[...], 0.0)
    logits = jax.lax.dot_general(
        h, w3t_ref[...], (((1,), (1,)), ((), ())),
        preferred_element_type=jnp.float32,
    )
    logits = logits + b3_ref[...]
    m = jnp.max(logits, axis=1, keepdims=True)
    e = jnp.exp(logits - m)
    p = e / jnp.sum(e, axis=1, keepdims=True)
    out_ref[...] = p.T


def kernel(x, edge_index, edge_weight, W1, b1, W2, b2, W3, b3):
    del edge_index, edge_weight
    f_in = x.shape[1]
    c = W2.shape[0]
    n_cls = W3.shape[1]
    w3t = W3.T

    grid = (pl.cdiv(_N, _BLK),)
    fixed = lambda i: (0, 0)
    fixed1 = lambda i: (0,)
    out_t = pl.pallas_call(
        _mlp_block,
        grid=grid,
        in_specs=[
            pl.BlockSpec((_BLK, f_in), lambda i: (i, 0)),
            pl.BlockSpec((f_in, c), fixed),
            pl.BlockSpec((c,), fixed1),
            pl.BlockSpec((c, c), fixed),
            pl.BlockSpec((c,), fixed1),
            pl.BlockSpec((n_cls, c), fixed),
            pl.BlockSpec((n_cls,), fixed1),
        ],
        out_specs=pl.BlockSpec((n_cls, _BLK), lambda i: (0, i)),
        out_shape=jax.ShapeDtypeStruct((n_cls, _N), jnp.float32),
        compiler_params=pltpu.CompilerParams(
            dimension_semantics=("arbitrary",),
        ),
    )(x, W1, b1, W2, b2, w3t, b3)
    return out_t.T


# speedup vs baseline: 2.9332x; 1.5262x over previous
import jax
import jax.numpy as jnp
from jax.experimental import pallas as pl
from jax.experimental.pallas import tpu as pltpu

_N = 10000
_BLK = 2560


def _mlp_block(x_ref, w1_ref, b1_ref, w2_ref, b2_ref, w3t_ref, b3_ref, out_ref):
    h = jnp.dot(x_ref[...], w1_ref[...], preferred_element_type=jnp.float32)
    h = jnp.maximum(h + b1_ref[...], 0.0)
    h = jnp.dot(h, w2_ref[...], preferred_element_type=jnp.float32)
    h = jnp.maximum(h + b2_ref[...], 0.0)
    logits_t = jax.lax.dot_general(
        w3t_ref[...], h, (((1,), (1,)), ((), ())),
        preferred_element_type=jnp.float32,
    )
    logits_t = logits_t + jnp.expand_dims(b3_ref[...], 1)
    m = jnp.max(logits_t, axis=0, keepdims=True)
    e = jnp.exp(logits_t - m)
    out_ref[...] = e / jnp.sum(e, axis=0, keepdims=True)


def kernel(x, edge_index, edge_weight, W1, b1, W2, b2, W3, b3):
    del edge_index, edge_weight
    f_in = x.shape[1]
    c = W2.shape[0]
    n_cls = W3.shape[1]
    w3t = W3.T

    grid = (pl.cdiv(_N, _BLK),)
    fixed = lambda i: (0, 0)
    fixed1 = lambda i: (0,)
    out_t = pl.pallas_call(
        _mlp_block,
        grid=grid,
        in_specs=[
            pl.BlockSpec((_BLK, f_in), lambda i: (i, 0)),
            pl.BlockSpec((f_in, c), fixed),
            pl.BlockSpec((c,), fixed1),
            pl.BlockSpec((c, c), fixed),
            pl.BlockSpec((c,), fixed1),
            pl.BlockSpec((n_cls, c), fixed),
            pl.BlockSpec((n_cls,), fixed1),
        ],
        out_specs=pl.BlockSpec((n_cls, _BLK), lambda i: (0, i)),
        out_shape=jax.ShapeDtypeStruct((n_cls, _N), jnp.float32),
        compiler_params=pltpu.CompilerParams(
            dimension_semantics=("arbitrary",),
        ),
    )(x, W1, b1, W2, b2, w3t, b3)
    return out_t.T


# speedup vs baseline: 3.3484x; 1.1415x over previous
import jax
import jax.numpy as jnp
from jax.experimental import pallas as pl
from jax.experimental.pallas import tpu as pltpu

_N = 10000
_BLK = 5120


def _mlp_block(x_ref, w1_ref, b1_ref, w2_ref, b2_ref, w3t_ref, b3_ref, out_ref):
    h = jnp.dot(x_ref[...], w1_ref[...], preferred_element_type=jnp.float32)
    h = jnp.maximum(h + b1_ref[...], 0.0)
    h = jnp.dot(h, w2_ref[...], preferred_element_type=jnp.float32)
    h = jnp.maximum(h + b2_ref[...], 0.0)
    logits_t = jax.lax.dot_general(
        w3t_ref[...], h, (((1,), (1,)), ((), ())),
        preferred_element_type=jnp.float32,
    )
    logits_t = logits_t + jnp.expand_dims(b3_ref[...], 1)
    m = jnp.max(logits_t, axis=0, keepdims=True)
    e = jnp.exp(logits_t - m)
    out_ref[...] = e / jnp.sum(e, axis=0, keepdims=True)


def kernel(x, edge_index, edge_weight, W1, b1, W2, b2, W3, b3):
    del edge_index, edge_weight
    f_in = x.shape[1]
    c = W2.shape[0]
    n_cls = W3.shape[1]
    w3t = W3.T

    grid = (pl.cdiv(_N, _BLK),)
    fixed = lambda i: (0, 0)
    fixed1 = lambda i: (0,)
    out_t = pl.pallas_call(
        _mlp_block,
        grid=grid,
        in_specs=[
            pl.BlockSpec((_BLK, f_in), lambda i: (i, 0)),
            pl.BlockSpec((f_in, c), fixed),
            pl.BlockSpec((c,), fixed1),
            pl.BlockSpec((c, c), fixed),
            pl.BlockSpec((c,), fixed1),
            pl.BlockSpec((n_cls, c), fixed),
            pl.BlockSpec((n_cls,), fixed1),
        ],
        out_specs=pl.BlockSpec((n_cls, _BLK), lambda i: (0, i)),
        out_shape=jax.ShapeDtypeStruct((n_cls, _N), jnp.float32),
        compiler_params=pltpu.CompilerParams(
            dimension_semantics=("arbitrary",),
        ),
    )(x, W1, b1, W2, b2, w3t, b3)
    return out_t.T
